# baseline (speedup 1.0000x reference)
import functools

import jax
import jax.numpy as jnp
from jax import lax
from jax.experimental import pallas as pl
from jax.experimental.pallas import tpu as pltpu
from jax.experimental.pallas import tpu_sc as plsc

_L = 16
_NW = 32
_CH = 128
_R = 8


def kernel(indices, table):
    B, S = indices.shape
    V, D = table.shape
    N = B * S
    assert N % (_NW * _CH) == 0 and D % _L == 0
    n_ch = N // (_NW * _CH)
    assert n_ch % _R == 0
    idx2d = indices.reshape(N // _CH, _CH)

    mesh = plsc.VectorSubcoreMesh(core_axis_name="c", subcore_axis_name="s")

    @functools.partial(
        pl.kernel,
        mesh=mesh,
        out_type=jax.ShapeDtypeStruct((N, D), jnp.float32),
        compiler_params=pltpu.CompilerParams(use_tc_tiling_on_sc=False),
        scratch_types=[
            pltpu.VMEM((n_ch, _CH), jnp.int32),
            pltpu.VMEM((_R, _CH, D), jnp.float32),
        ]
        + [pltpu.SemaphoreType.DMA] * _R
        + [pltpu.SemaphoreType.DMA] * _R,
    )
    def _emb(idx_hbm, table_hbm, out_hbm, idx_v, rows_v, *sems):
        gsems = sems[:_R]
        wsems = sems[_R:]
        wid = lax.axis_index("s") * 2 + lax.axis_index("c")
        pltpu.sync_copy(idx_hbm.at[pl.ds(wid * n_ch, n_ch)], idx_v)

        def start_gather(j, b):
            pltpu.async_copy(table_hbm.at[idx_v.at[j]], rows_v.at[b], gsems[b])

        def wait_gather(j, b):
            pltpu.make_async_copy(
                table_hbm.at[idx_v.at[j]], rows_v.at[b], gsems[b]
            ).wait()

        def out_slice(j):
            return out_hbm.at[pl.ds((wid * n_ch + j) * _CH, _CH)]

        def process(j, b):
            def cnt_group(i, accv):
                iv = idx_v[j, pl.ds(i * _L, _L)]
                return accv + (1 - jnp.minimum(iv, 1))

            accv = lax.fori_loop(
                0, _CH // _L, cnt_group, jnp.zeros((_L,), jnp.int32)
            )
            nzero = accv[0]
            for _k in range(1, _L):
                nzero = nzero + accv[_k]

            @pl.when(nzero > 0)
            def _fix():
                zeros = jnp.zeros((_L,), jnp.float32)

                def fix_group(i, carry2):
                    iv = idx_v[j, pl.ds(i * _L, _L)]
                    for l in range(_L):
                        val = iv[l]

                        @pl.when(val == 0)
                        def _zrow(l=l):
                            row = i * _L + l
                            for c in range(D // _L):
                                rows_v[b, row, pl.ds(c * _L, _L)] = zeros

                    return carry2

                lax.fori_loop(0, _CH // _L, fix_group, 0)

            pltpu.async_copy(rows_v.at[b], out_slice(j), wsems[b])

        def wait_write(j, b):
            pltpu.make_async_copy(
                rows_v.at[b], out_slice(j), wsems[b]
            ).wait()

        for b in range(_R):
            start_gather(b, b)

        n_steps = n_ch // _R

        def step_body(step, carry):
            for b in range(_R):
                j = step * _R + b
                wait_gather(j, b)
                process(j, b)

                @pl.when(step < n_steps - 1)
                def _next(j=j, b=b):
                    wait_write(j, b)
                    start_gather(j + _R, b)

            return carry

        lax.fori_loop(0, n_steps, step_body, 0)

        for b in range(_R):
            wait_write(n_ch - _R + b, b)

    out = _emb(idx2d, table)
    return out.reshape(B, S, D)


# speedup vs baseline: 1.0075x; 1.0075x over previous
import functools

import jax
import jax.numpy as jnp
from jax import lax
from jax.experimental import pallas as pl
from jax.experimental.pallas import tpu as pltpu
from jax.experimental.pallas import tpu_sc as plsc

_L = 16
_NW = 32
_IB = 128
_G = 2
_CH = _IB * _G
_R = 4


def kernel(indices, table):
    B, S = indices.shape
    V, D = table.shape
    N = B * S
    assert N % (_NW * _CH) == 0 and D % _L == 0
    n_ch = N // (_NW * _CH)
    n_ib = n_ch * _G
    assert n_ch % _R == 0
    idx2d = indices.reshape(N // _IB, _IB)

    mesh = plsc.VectorSubcoreMesh(core_axis_name="c", subcore_axis_name="s")

    @functools.partial(
        pl.kernel,
        mesh=mesh,
        out_type=jax.ShapeDtypeStruct((N, D), jnp.float32),
        compiler_params=pltpu.CompilerParams(use_tc_tiling_on_sc=False),
        scratch_types=[
            pltpu.VMEM((n_ib, _IB), jnp.int32),
            pltpu.VMEM((_R, _CH, D), jnp.float32),
        ]
        + [pltpu.SemaphoreType.DMA] * _R
        + [pltpu.SemaphoreType.DMA] * _R,
    )
    def _emb(idx_hbm, table_hbm, out_hbm, idx_v, rows_v, *sems):
        gsems = sems[:_R]
        wsems = sems[_R:]
        wid = lax.axis_index("s") * 2 + lax.axis_index("c")
        pltpu.sync_copy(idx_hbm.at[pl.ds(wid * n_ib, n_ib)], idx_v)

        def start_gather(j, b):
            for g in range(_G):
                pltpu.async_copy(
                    table_hbm.at[idx_v.at[j * _G + g]],
                    rows_v.at[b].at[pl.ds(g * _IB, _IB)],
                    gsems[b],
                )

        def wait_gather(j, b):
            for g in range(_G):
                pltpu.make_async_copy(
                    table_hbm.at[idx_v.at[j * _G + g]],
                    rows_v.at[b].at[pl.ds(g * _IB, _IB)],
                    gsems[b],
                ).wait()

        def out_slice(j):
            return out_hbm.at[pl.ds((wid * n_ch + j) * _CH, _CH)]

        def process(j, b):
            def cnt_group(i, accv):
                iv = idx_v[j * _G + i // (_IB // _L),
                           pl.ds((i % (_IB // _L)) * _L, _L)]
                return accv + (1 - jnp.minimum(iv, 1))

            accv = lax.fori_loop(
                0, _CH // _L, cnt_group, jnp.zeros((_L,), jnp.int32)
            )
            nzero = accv[0]
            for _k in range(1, _L):
                nzero = nzero + accv[_k]

            @pl.when(nzero > 0)
            def _fix():
                zeros = jnp.zeros((_L,), jnp.float32)

                def fix_group(i, carry2):
                    iv = idx_v[j * _G + i // (_IB // _L),
                               pl.ds((i % (_IB // _L)) * _L, _L)]
                    for l in range(_L):
                        val = iv[l]

                        @pl.when(val == 0)
                        def _zrow(l=l):
                            row = i * _L + l
                            for c in range(D // _L):
                                rows_v[b, row, pl.ds(c * _L, _L)] = zeros

                    return carry2

                lax.fori_loop(0, _CH // _L, fix_group, 0)

            pltpu.async_copy(rows_v.at[b], out_slice(j), wsems[b])

        def wait_write(j, b):
            pltpu.make_async_copy(
                rows_v.at[b], out_slice(j), wsems[b]
            ).wait()

        for b in range(_R):
            start_gather(b, b)

        n_steps = n_ch // _R

        def step_body(step, carry):
            for b in range(_R):
                j = step * _R + b
                wait_gather(j, b)
                process(j, b)

                @pl.when(step < n_steps - 1)
                def _next(j=j, b=b):
                    wait_write(j, b)
                    start_gather(j + _R, b)

            return carry

        lax.fori_loop(0, n_steps, step_body, 0)

        for b in range(_R):
            wait_write(n_ch - _R + b, b)

    out = _emb(idx2d, table)
    return out.reshape(B, S, D)


# speedup vs baseline: 1.0460x; 1.0383x over previous
import functools

import jax
import jax.numpy as jnp
from jax import lax
from jax.experimental import pallas as pl
from jax.experimental.pallas import tpu as pltpu
from jax.experimental.pallas import tpu_sc as plsc

_L = 16
_NW = 32
_IB = 128
_G = 2
_CH = _IB * _G
_R = 4


def kernel(indices, table):
    indices = jnp.transpose(indices)
    S, B = indices.shape
    V, D = table.shape
    N = B * S
    assert N % (_NW * _CH) == 0 and D % _L == 0
    n_ch = N // (_NW * _CH)
    n_ib = n_ch * _G
    assert n_ch % _R == 0
    idx2d = indices.reshape(N // _IB, _IB)

    mesh = plsc.VectorSubcoreMesh(core_axis_name="c", subcore_axis_name="s")

    @functools.partial(
        pl.kernel,
        mesh=mesh,
        out_type=jax.ShapeDtypeStruct((N, D), jnp.float32),
        compiler_params=pltpu.CompilerParams(use_tc_tiling_on_sc=False),
        scratch_types=[
            pltpu.VMEM((n_ib, _IB), jnp.int32),
            pltpu.VMEM((_R, _CH, D), jnp.float32),
        ]
        + [pltpu.SemaphoreType.DMA] * _R
        + [pltpu.SemaphoreType.DMA] * _R,
    )
    def _emb(idx_hbm, table_hbm, out_hbm, idx_v, rows_v, *sems):
        gsems = sems[:_R]
        wsems = sems[_R:]
        wid = lax.axis_index("s") * 2 + lax.axis_index("c")
        pltpu.sync_copy(idx_hbm.at[pl.ds(wid * n_ib, n_ib)], idx_v)

        def start_gather(j, b):
            for g in range(_G):
                pltpu.async_copy(
                    table_hbm.at[idx_v.at[j * _G + g]],
                    rows_v.at[b].at[pl.ds(g * _IB, _IB)],
                    gsems[b],
                )

        def wait_gather(j, b):
            for g in range(_G):
                pltpu.make_async_copy(
                    table_hbm.at[idx_v.at[j * _G + g]],
                    rows_v.at[b].at[pl.ds(g * _IB, _IB)],
                    gsems[b],
                ).wait()

        def out_slice(j):
            return out_hbm.at[pl.ds((wid * n_ch + j) * _CH, _CH)]

        def process(j, b):
            def cnt_group(i, accv):
                iv = idx_v[j * _G + i // (_IB // _L),
                           pl.ds((i % (_IB // _L)) * _L, _L)]
                return accv + (1 - jnp.minimum(iv, 1))

            accv = lax.fori_loop(
                0, _CH // _L, cnt_group, jnp.zeros((_L,), jnp.int32)
            )
            nzero = accv[0]
            for _k in range(1, _L):
                nzero = nzero + accv[_k]

            @pl.when(nzero > 0)
            def _fix():
                zeros = jnp.zeros((_L,), jnp.float32)

                def fix_group(i, carry2):
                    iv = idx_v[j * _G + i // (_IB // _L),
                               pl.ds((i % (_IB // _L)) * _L, _L)]
                    for l in range(_L):
                        val = iv[l]

                        @pl.when(val == 0)
                        def _zrow(l=l):
                            row = i * _L + l
                            for c in range(D // _L):
                                rows_v[b, row, pl.ds(c * _L, _L)] = zeros

                    return carry2

                lax.fori_loop(0, _CH // _L, fix_group, 0)

            pltpu.async_copy(rows_v.at[b], out_slice(j), wsems[b])

        def wait_write(j, b):
            pltpu.make_async_copy(
                rows_v.at[b], out_slice(j), wsems[b]
            ).wait()

        for b in range(_R):
            start_gather(b, b)

        n_steps = n_ch // _R

        def step_body(step, carry):
            for b in range(_R):
                j = step * _R + b
                wait_gather(j, b)
                process(j, b)

                @pl.when(step < n_steps - 1)
                def _next(j=j, b=b):
                    wait_write(j, b)
                    start_gather(j + _R, b)

            return carry

        lax.fori_loop(0, n_steps, step_body, 0)

        for b in range(_R):
            wait_write(n_ch - _R + b, b)

    out = _emb(idx2d, table)
    return jnp.transpose(out.reshape(S, B, D), (1, 0, 2))
